# Initial kernel scaffold; baseline (speedup 1.0000x reference)
import jax
import jax.numpy as jnp
from jax.experimental import pallas as pl


def kernel(x, edge_index, W1, b1, W2, b2, Wo, bo):
    raise NotImplementedError("write your pallas kernel here")


# speedup vs baseline: 12.5773x; 12.5773x over previous
import functools

import jax
import jax.numpy as jnp
from jax import lax
from jax.experimental import pallas as pl
from jax.experimental.pallas import tpu as pltpu
from jax.experimental.pallas import tpu_sc as plsc

NC = 2
NS = 16
NW = NC * NS
K_CH = 128


def _make_agg(n_rows, d, n_chunks):
  mesh = plsc.VectorSubcoreMesh(core_axis_name="c", subcore_axis_name="s")
  rows_per_tile = n_rows // NS

  @functools.partial(
      pl.kernel,
      mesh=mesh,
      out_type=jax.ShapeDtypeStruct((NC, n_rows, d), jnp.float32),
      scratch_types=[
          pltpu.VMEM((n_chunks, K_CH), jnp.int32),
          pltpu.VMEM((n_chunks, K_CH), jnp.int32),
          pltpu.VMEM((K_CH, d), jnp.float32),
          pltpu.VMEM_SHARED((n_rows, d), jnp.float32),
          pltpu.SemaphoreType.DMA,
      ],
  )
  def agg(table_hbm, src_hbm, dst_hbm, zeros_hbm, out_hbm,
          src_v, dst_v, buf, acc, sem):
    cid = lax.axis_index("c")
    sid = lax.axis_index("s")
    wid = sid * NC + cid
    pltpu.sync_copy(src_hbm.at[wid], src_v)
    pltpu.sync_copy(dst_hbm.at[wid], dst_v)

    @pl.when(sid == 0)
    def _zero():
      pltpu.sync_copy(zeros_hbm, acc)

    plsc.subcore_barrier()

    def body(ci, carry):
      pltpu.async_copy(table_hbm.at[src_v.at[ci]], buf, sem).wait()
      pltpu.sync_copy(buf, acc.at[dst_v.at[ci]], add=True)
      return carry

    lax.fori_loop(0, n_chunks, body, 0)
    plsc.subcore_barrier()

    r0 = sid * rows_per_tile
    pltpu.sync_copy(acc.at[pl.ds(r0, rows_per_tile)],
                    out_hbm.at[cid, pl.ds(r0, rows_per_tile)])

  return agg


def _make_deg(n_rows, d, n_chunks):
  mesh = plsc.VectorSubcoreMesh(core_axis_name="c", subcore_axis_name="s")
  rows_per_tile = n_rows // NS

  @functools.partial(
      pl.kernel,
      mesh=mesh,
      out_type=jax.ShapeDtypeStruct((NC, n_rows, d), jnp.float32),
      scratch_types=[
          pltpu.VMEM((n_chunks, K_CH), jnp.int32),
          pltpu.VMEM((K_CH, d), jnp.float32),
          pltpu.VMEM_SHARED((n_rows, d), jnp.float32),
      ],
  )
  def deg(dst_hbm, zeros_hbm, out_hbm, dst_v, buf, acc):
    cid = lax.axis_index("c")
    sid = lax.axis_index("s")
    wid = sid * NC + cid
    pltpu.sync_copy(dst_hbm.at[wid], dst_v)

    ones = jnp.ones((16,), jnp.float32)

    def fbody(i, carry):
      buf[i // (d // 16), pl.ds((i % (d // 16)) * 16, 16)] = ones
      return carry

    lax.fori_loop(0, K_CH * d // 16, fbody, 0)

    @pl.when(sid == 0)
    def _zero():
      pltpu.sync_copy(zeros_hbm, acc)

    plsc.subcore_barrier()

    def body(ci, carry):
      pltpu.sync_copy(buf, acc.at[dst_v.at[ci]], add=True)
      return carry

    lax.fori_loop(0, n_chunks, body, 0)
    plsc.subcore_barrier()

    r0 = sid * rows_per_tile
    pltpu.sync_copy(acc.at[pl.ds(r0, rows_per_tile)],
                    out_hbm.at[cid, pl.ds(r0, rows_per_tile)])

  return deg


def _dinv_of(degp_ref):
  deg = degp_ref[0, :, 0] + degp_ref[1, :, 0] + 1.0
  return lax.rsqrt(deg)


def _scale_in_body(x_ref, degp_ref, w_ref, o_ref):
  dinv = _dinv_of(degp_ref)
  h = jnp.dot(x_ref[...], w_ref[...], preferred_element_type=jnp.float32)
  o_ref[...] = h * dinv[:, None]


def _combine_mm_body(aggp_ref, hs_ref, degp_ref, b_ref, w_ref, o_ref):
  dinv = _dinv_of(degp_ref)
  t = (aggp_ref[0] + aggp_ref[1] + hs_ref[...]) * dinv[:, None] + b_ref[...]
  h = jnp.maximum(t, 0.0)
  o_ref[...] = jnp.dot(h, w_ref[...],
                       preferred_element_type=jnp.float32) * dinv[:, None]


def _combine_head_body(aggp_ref, hs_ref, degp_ref, b_ref, wo_ref, bo_ref,
                       o_ref):
  dinv = _dinv_of(degp_ref)
  t = (aggp_ref[0] + aggp_ref[1] + hs_ref[...]) * dinv[:, None] + b_ref[...]
  h = jnp.maximum(t, 0.0)
  o_ref[...] = jnp.dot(h, wo_ref[...],
                       preferred_element_type=jnp.float32) + bo_ref[...]


def _softmax_body(n_valid, l_ref, o_ref):
  l = l_ref[...]
  ridx = lax.broadcasted_iota(jnp.int32, l.shape, 0)
  valid = ridx < n_valid
  lm = jnp.where(valid, l, -1e30)
  m = jnp.max(lm)
  e = jnp.where(valid, jnp.exp(lm - m), 0.0)
  o_ref[...] = e / jnp.sum(e)


def kernel(x, edge_index, W1, b1, W2, b2, Wo, bo):
  n, d = x.shape
  h_dim = W1.shape[1]
  e = edge_index.shape[1]

  bm = 1024
  npad = ((n + bm - 1) // bm) * bm
  epad = ((e + NW * K_CH - 1) // (NW * K_CH)) * (NW * K_CH)
  n_chunks = epad // (NW * K_CH)

  pad = jnp.full((epad - e,), n, dtype=edge_index.dtype)
  src_flat = jnp.concatenate([edge_index[0], pad])
  dst_flat = jnp.concatenate([edge_index[1], pad])
  src = src_flat.reshape(NW, n_chunks, K_CH)
  dst = dst_flat.reshape(NW, n_chunks, K_CH)

  x_p = jnp.pad(x, ((0, npad - n), (0, 0)))
  zd = jnp.zeros((npad, h_dim), jnp.float32)
  z16 = jnp.zeros((npad, 16), jnp.float32)

  aggd = _make_agg(npad, h_dim, n_chunks)
  degk = _make_deg(npad, 16, n_chunks)

  degp = degk(dst, z16)

  grid = npad // bm
  full_w = pl.BlockSpec((d, h_dim), lambda i: (0, 0))
  blk_rows = pl.BlockSpec((bm, h_dim), lambda i: (i, 0))
  blk_deg = pl.BlockSpec((NC, bm, 16), lambda i: (0, i, 0))
  blk_agg = pl.BlockSpec((NC, bm, h_dim), lambda i: (0, i, 0))
  blk_b = pl.BlockSpec((1, h_dim), lambda i: (0, 0))

  hs1 = pl.pallas_call(
      _scale_in_body,
      grid=(grid,),
      in_specs=[pl.BlockSpec((bm, d), lambda i: (i, 0)), blk_deg, full_w],
      out_specs=blk_rows,
      out_shape=jax.ShapeDtypeStruct((npad, h_dim), jnp.float32),
  )(x_p, degp, W1)

  aggp1 = aggd(hs1, src, dst, zd)

  hs2 = pl.pallas_call(
      _combine_mm_body,
      grid=(grid,),
      in_specs=[blk_agg, blk_rows, blk_deg, blk_b,
                pl.BlockSpec((h_dim, h_dim), lambda i: (0, 0))],
      out_specs=blk_rows,
      out_shape=jax.ShapeDtypeStruct((npad, h_dim), jnp.float32),
  )(aggp1, hs1, degp, b1.reshape(1, h_dim), W2)

  aggp2 = aggd(hs2, src, dst, zd)

  logits = pl.pallas_call(
      _combine_head_body,
      grid=(grid,),
      in_specs=[blk_agg, blk_rows, blk_deg, blk_b,
                pl.BlockSpec((h_dim, 1), lambda i: (0, 0)),
                pl.BlockSpec((1, 1), lambda i: (0, 0))],
      out_specs=pl.BlockSpec((bm, 1), lambda i: (i, 0)),
      out_shape=jax.ShapeDtypeStruct((npad, 1), jnp.float32),
  )(aggp2, hs2, degp, b2.reshape(1, h_dim), Wo, bo.reshape(1, 1))

  probs = pl.pallas_call(
      functools.partial(_softmax_body, n),
      in_specs=[pl.BlockSpec((npad, 1), lambda: (0, 0))],
      out_specs=pl.BlockSpec((npad, 1), lambda: (0, 0)),
      out_shape=jax.ShapeDtypeStruct((npad, 1), jnp.float32),
  )(logits)

  return probs[:n, 0]
